# baseline (speedup 1.0000x reference)
import functools

import numpy as np

import jax
import jax.numpy as jnp
from jax import lax
from jax.experimental import pallas as pl
from jax.experimental.pallas import tpu as pltpu
from jax.experimental.pallas import tpu_sc as plsc

RES = 100
D = 64
N_ROWS = 16384
N_COLS = 20
NJP = N_COLS // 2
B = N_ROWS * N_COLS
P = B // 2
NC = 2
NS = 16
NW = NC * NS
RPW = N_ROWS // NW
PPW = P // NW
C = 256
NBUF = 2
UNROLL = 4

_MAGIC = np.float32(2.0 ** 23)
_LO = np.float32(1.0 / RES)
_ONE = np.float32(1.0)
_RESF = np.float32(RES)
_P101 = np.float32(101.0)


def _round_clip(v):
    xc = jnp.minimum(jnp.maximum(v, _LO), _ONE)
    return (xc * _RESF + _MAGIC) - _MAGIC


def _body(xe_hbm, xo_hbm, pe2_hbm, out_hbm, xe_v, xo_v, idx_v, *rest):
    rows = rest[:NBUF]
    gsem = rest[NBUF:2 * NBUF]
    ssem = rest[2 * NBUF:3 * NBUF]
    wid = lax.axis_index("s") * NC + lax.axis_index("c")
    base = wid * PPW
    r0 = wid * RPW

    pltpu.sync_copy(xe_hbm.at[pl.ds(base, PPW)], xe_v)
    pltpu.sync_copy(xo_hbm.at[pl.ds(base, PPW)], xo_v)

    iota10 = lax.iota(jnp.int32, 16) * np.int32(NJP)

    def cidx(s, carry):
        for u in range(UNROLL):
            t = s * UNROLL + u
            jp = t // (RPW // 16)
            tt = t % (RPW // 16)
            offs = iota10 + (tt * (16 * NJP) + jp)
            ge = plsc.load_gather(xe_v, [offs])
            go = plsc.load_gather(xo_v, [offs])
            pid = _round_clip(ge) * _RESF + _round_clip(go) - _P101
            idx_v[pl.ds(jp * RPW + tt * 16, 16)] = pid.astype(jnp.int32)
        return carry

    lax.fori_loop(0, PPW // (16 * UNROLL), cidx, 0)

    def fire_gather(g, h, b):
        pltpu.async_copy(
            pe2_hbm.at[idx_v.at[pl.ds(g * RPW + h * C, C)]], rows[b], gsem[b])

    def wait_gather(b):
        pltpu.make_async_copy(
            pe2_hbm.at[idx_v.at[pl.ds(0, C)]], rows[b], gsem[b]).wait()

    def fire_scatter(g, h, b):
        pltpu.async_copy(
            rows[b], out_hbm.at[g, pl.ds(r0 + h * C, C)], ssem[b])

    def wait_scatter(b):
        pltpu.make_async_copy(
            rows[b], out_hbm.at[0, pl.ds(0, C)], ssem[b]).wait()

    fire_gather(0, 0, 0)
    fire_gather(0, 1, 1)

    def group(g, carry):
        for b in range(NBUF):
            wait_gather(b)
            fire_scatter(g, b, b)
        for b in range(NBUF):
            wait_scatter(b)
            fire_gather(g + 1, b, b)
        return carry

    lax.fori_loop(0, NJP - 1, group, 0)

    for b in range(NBUF):
        wait_gather(b)
        fire_scatter(NJP - 1, b, b)
    for b in range(NBUF):
        wait_scatter(b)


@jax.jit
def _emb(xe, xo, pe2):
    mesh = plsc.VectorSubcoreMesh(core_axis_name="c", subcore_axis_name="s")
    k = pl.kernel(
        _body,
        out_type=jax.ShapeDtypeStruct((NJP, N_ROWS, 2 * D), jnp.float32),
        mesh=mesh,
        scratch_types=(
            [
                pltpu.VMEM((PPW,), jnp.float32),
                pltpu.VMEM((PPW,), jnp.float32),
                pltpu.VMEM((PPW,), jnp.int32),
            ]
            + [pltpu.VMEM((C, 2 * D), jnp.float32) for _ in range(NBUF)]
            + [pltpu.SemaphoreType.DMA for _ in range(2 * NBUF)]
        ),
        compiler_params=pltpu.CompilerParams(
            use_tc_tiling_on_sc=True, needs_layout_passes=False),
    )
    return k(xe, xo, pe2)


def _xpose_body(in_ref, out_ref):
    a = in_ref[0]
    out_ref[...] = a.T.reshape(2, D, 128)


@jax.jit
def _xpose(mid):
    return pl.pallas_call(
        _xpose_body,
        grid=(NJP, N_ROWS // 128),
        in_specs=[pl.BlockSpec((1, 128, 128), lambda jp, rb: (jp, rb, 0))],
        out_specs=pl.BlockSpec((2, D, 128), lambda jp, rb: (jp, 0, rb)),
        out_shape=jax.ShapeDtypeStruct((N_COLS, D, N_ROWS), jnp.float32),
    )(mid)


def kernel(x, pe):
    xf = x.reshape(B)
    pe2 = jnp.concatenate(
        [
            jnp.repeat(pe, RES, axis=0),
            jnp.tile(pe, (RES, 1)),
        ],
        axis=1,
    )
    mid = _emb(xf[0::2], xf[1::2], pe2)
    out = _xpose(mid)
    return jnp.transpose(out, (2, 0, 1))


# speedup vs baseline: 3.2443x; 3.2443x over previous
import functools

import numpy as np

import jax
import jax.numpy as jnp
from jax import lax
from jax.experimental import pallas as pl
from jax.experimental.pallas import tpu as pltpu
from jax.experimental.pallas import tpu_sc as plsc

RES = 100
D = 64
N_ROWS = 16384
N_COLS = 20
NJP = N_COLS // 2
B = N_ROWS * N_COLS
P = B // 2
NC = 2
NS = 16
NW = NC * NS
RPW = N_ROWS // NW
PPW = P // NW
C = 256
NBUF = 2
UNROLL = 4

_MAGIC = np.float32(2.0 ** 23)
_LO = np.float32(1.0 / RES)
_ONE = np.float32(1.0)
_RESF = np.float32(RES)
_P101 = np.float32(101.0)


def _round_clip(v):
    xc = jnp.minimum(jnp.maximum(v, _LO), _ONE)
    return (xc * _RESF + _MAGIC) - _MAGIC


def _body(xe_hbm, xo_hbm, pe2_hbm, out_hbm, xe_v, xo_v, idx_v, *rest):
    rows = rest[:NBUF]
    gsem = rest[NBUF:2 * NBUF]
    ssem = rest[2 * NBUF:3 * NBUF]
    wid = lax.axis_index("s") * NC + lax.axis_index("c")
    base = wid * PPW
    r0 = wid * RPW

    pltpu.sync_copy(xe_hbm.at[pl.ds(base, PPW)], xe_v)
    pltpu.sync_copy(xo_hbm.at[pl.ds(base, PPW)], xo_v)

    iota10 = lax.iota(jnp.int32, 16) * np.int32(NJP)

    def cidx(s, carry):
        for u in range(UNROLL):
            t = s * UNROLL + u
            jp = t // (RPW // 16)
            tt = t % (RPW // 16)
            offs = iota10 + (tt * (16 * NJP) + jp)
            ge = plsc.load_gather(xe_v, [offs])
            go = plsc.load_gather(xo_v, [offs])
            pid = _round_clip(ge) * _RESF + _round_clip(go) - _P101
            idx_v[pl.ds(jp * RPW + tt * 16, 16)] = pid.astype(jnp.int32)
        return carry

    lax.fori_loop(0, PPW // (16 * UNROLL), cidx, 0)

    def fire_gather(g, h, b):
        pltpu.async_copy(
            pe2_hbm.at[idx_v.at[pl.ds(g * RPW + h * C, C)]], rows[b], gsem[b])

    def wait_gather(b):
        pltpu.make_async_copy(
            pe2_hbm.at[idx_v.at[pl.ds(0, C)]], rows[b], gsem[b]).wait()

    def fire_scatter(g, h, b):
        pltpu.async_copy(
            rows[b], out_hbm.at[g, pl.ds(r0 + h * C, C)], ssem[b])

    def wait_scatter(b):
        pltpu.make_async_copy(
            rows[b], out_hbm.at[0, pl.ds(0, C)], ssem[b]).wait()

    fire_gather(0, 0, 0)
    fire_gather(0, 1, 1)

    def group(g, carry):
        for b in range(NBUF):
            wait_gather(b)
            fire_scatter(g, b, b)
        for b in range(NBUF):
            wait_scatter(b)
            fire_gather(g + 1, b, b)
        return carry

    lax.fori_loop(0, NJP - 1, group, 0)

    for b in range(NBUF):
        wait_gather(b)
        fire_scatter(NJP - 1, b, b)
    for b in range(NBUF):
        wait_scatter(b)


@jax.jit
def _emb(xe, xo, pe2):
    mesh = plsc.VectorSubcoreMesh(core_axis_name="c", subcore_axis_name="s")
    k = pl.kernel(
        _body,
        out_type=jax.ShapeDtypeStruct((NJP, N_ROWS, 2 * D), jnp.float32),
        mesh=mesh,
        scratch_types=(
            [
                pltpu.VMEM((PPW,), jnp.float32),
                pltpu.VMEM((PPW,), jnp.float32),
                pltpu.VMEM((PPW,), jnp.int32),
            ]
            + [pltpu.VMEM((C, 2 * D), jnp.float32) for _ in range(NBUF)]
            + [pltpu.SemaphoreType.DMA for _ in range(2 * NBUF)]
        ),
        compiler_params=pltpu.CompilerParams(
            use_tc_tiling_on_sc=True, needs_layout_passes=False),
    )
    return k(xe, xo, pe2)


XR = 2048


def _xpose_body(in_ref, out_ref):
    a = in_ref[0]
    for sub in range(XR // 128):
        blk = a[sub * 128:(sub + 1) * 128, :]
        out_ref[:, :, sub * 128:(sub + 1) * 128] = blk.T.reshape(2, D, 128)


@jax.jit
def _xpose(mid):
    return pl.pallas_call(
        _xpose_body,
        grid=(NJP, N_ROWS // XR),
        in_specs=[pl.BlockSpec((1, XR, 128), lambda jp, rb: (jp, rb, 0))],
        out_specs=pl.BlockSpec((2, D, XR), lambda jp, rb: (jp, 0, rb)),
        out_shape=jax.ShapeDtypeStruct((N_COLS, D, N_ROWS), jnp.float32),
    )(mid)


def kernel(x, pe):
    xf = x.reshape(B)
    pe2 = jnp.concatenate(
        [
            jnp.repeat(pe, RES, axis=0),
            jnp.tile(pe, (RES, 1)),
        ],
        axis=1,
    )
    mid = _emb(xf[0::2], xf[1::2], pe2)
    out = _xpose(mid)
    return jnp.transpose(out, (2, 0, 1))


# speedup vs baseline: 3.5826x; 1.1043x over previous
import functools

import numpy as np

import jax
import jax.numpy as jnp
from jax import lax
from jax.experimental import pallas as pl
from jax.experimental.pallas import tpu as pltpu
from jax.experimental.pallas import tpu_sc as plsc

RES = 100
D = 64
N_ROWS = 16384
N_COLS = 20
NJP = N_COLS // 2
B = N_ROWS * N_COLS
P = B // 2
NC = 2
NS = 16
NW = NC * NS
RPW = N_ROWS // NW
PPW = P // NW
C = 256
NBUF = 2
UNROLL = 4

_MAGIC = np.float32(2.0 ** 23)
_LO = np.float32(1.0 / RES)
_ONE = np.float32(1.0)
_RESF = np.float32(RES)
_P101 = np.float32(101.0)


def _round_clip(v):
    xc = jnp.minimum(jnp.maximum(v, _LO), _ONE)
    return (xc * _RESF + _MAGIC) - _MAGIC


def _body(xe_hbm, xo_hbm, pe2_hbm, out_hbm, xe_v, xo_v, idx_v, *rest):
    rows = rest[:NBUF]
    gsem = rest[NBUF:2 * NBUF]
    ssem = rest[2 * NBUF:3 * NBUF]
    wid = lax.axis_index("s") * NC + lax.axis_index("c")
    base = wid * PPW
    r0 = wid * RPW

    pltpu.sync_copy(xe_hbm.at[pl.ds(base, PPW)], xe_v)
    pltpu.sync_copy(xo_hbm.at[pl.ds(base, PPW)], xo_v)

    iota10 = lax.iota(jnp.int32, 16) * np.int32(NJP)

    def cidx(s, carry):
        for u in range(UNROLL):
            t = s * UNROLL + u
            jp = t // (RPW // 16)
            tt = t % (RPW // 16)
            offs = iota10 + (tt * (16 * NJP) + jp)
            ge = plsc.load_gather(xe_v, [offs])
            go = plsc.load_gather(xo_v, [offs])
            pid = _round_clip(ge) * _RESF + _round_clip(go) - _P101
            idx_v[pl.ds(jp * RPW + tt * 16, 16)] = pid.astype(jnp.int32)
        return carry

    lax.fori_loop(0, PPW // (16 * UNROLL), cidx, 0)

    def fire_gather(g, h, b):
        pltpu.async_copy(
            pe2_hbm.at[idx_v.at[pl.ds(g * RPW + h * C, C)]], rows[b], gsem[b])

    def wait_gather(b):
        pltpu.make_async_copy(
            pe2_hbm.at[idx_v.at[pl.ds(0, C)]], rows[b], gsem[b]).wait()

    def fire_scatter(g, h, b):
        pltpu.async_copy(
            rows[b], out_hbm.at[g, pl.ds(r0 + h * C, C)], ssem[b])

    def wait_scatter(b):
        pltpu.make_async_copy(
            rows[b], out_hbm.at[0, pl.ds(0, C)], ssem[b]).wait()

    fire_gather(0, 0, 0)
    fire_gather(0, 1, 1)

    def group(g, carry):
        for b in range(NBUF):
            wait_gather(b)
            fire_scatter(g, b, b)
        for b in range(NBUF):
            wait_scatter(b)
            fire_gather(g + 1, b, b)
        return carry

    lax.fori_loop(0, NJP - 1, group, 0)

    for b in range(NBUF):
        wait_gather(b)
        fire_scatter(NJP - 1, b, b)
    for b in range(NBUF):
        wait_scatter(b)


@jax.jit
def _emb(xe, xo, pe2):
    mesh = plsc.VectorSubcoreMesh(core_axis_name="c", subcore_axis_name="s")
    k = pl.kernel(
        _body,
        out_type=jax.ShapeDtypeStruct((NJP, N_ROWS, 2 * D), jnp.float32),
        mesh=mesh,
        scratch_types=(
            [
                pltpu.VMEM((PPW,), jnp.float32),
                pltpu.VMEM((PPW,), jnp.float32),
                pltpu.VMEM((PPW,), jnp.int32),
            ]
            + [pltpu.VMEM((C, 2 * D), jnp.float32) for _ in range(NBUF)]
            + [pltpu.SemaphoreType.DMA for _ in range(2 * NBUF)]
        ),
        compiler_params=pltpu.CompilerParams(
            use_tc_tiling_on_sc=True, needs_layout_passes=False),
    )
    return k(xe, xo, pe2)


XR = 4096


def _xpose_body(in_ref, out_ref):
    a = in_ref[0]
    for sub in range(XR // 128):
        blk = a[sub * 128:(sub + 1) * 128, :]
        out_ref[:, :, sub * 128:(sub + 1) * 128] = blk.T.reshape(2, D, 128)


@jax.jit
def _xpose(mid):
    return pl.pallas_call(
        _xpose_body,
        grid=(NJP, N_ROWS // XR),
        in_specs=[pl.BlockSpec((1, XR, 128), lambda jp, rb: (jp, rb, 0))],
        out_specs=pl.BlockSpec((2, D, XR), lambda jp, rb: (jp, 0, rb)),
        out_shape=jax.ShapeDtypeStruct((N_COLS, D, N_ROWS), jnp.float32),
    )(mid)


def kernel(x, pe):
    xf = x.reshape(B)
    pe2 = jnp.concatenate(
        [
            jnp.repeat(pe, RES, axis=0),
            jnp.tile(pe, (RES, 1)),
        ],
        axis=1,
    )
    mid = _emb(xf[0::2], xf[1::2], pe2)
    out = _xpose(mid)
    return jnp.transpose(out, (2, 0, 1))


# speedup vs baseline: 3.7227x; 1.0391x over previous
import functools

import numpy as np

import jax
import jax.numpy as jnp
from jax import lax
from jax.experimental import pallas as pl
from jax.experimental.pallas import tpu as pltpu
from jax.experimental.pallas import tpu_sc as plsc

RES = 100
D = 64
N_ROWS = 16384
N_COLS = 20
NJP = N_COLS // 2
B = N_ROWS * N_COLS
P = B // 2
NC = 2
NS = 16
NW = NC * NS
RPW = N_ROWS // NW
PPW = P // NW
C = 256
NBUF = 2
UNROLL = 4

_MAGIC = np.float32(2.0 ** 23)
_LO = np.float32(1.0 / RES)
_ONE = np.float32(1.0)
_RESF = np.float32(RES)
_P101 = np.float32(101.0)


def _round_clip(v):
    xc = jnp.minimum(jnp.maximum(v, _LO), _ONE)
    return (xc * _RESF + _MAGIC) - _MAGIC


def _body(xe_hbm, xo_hbm, pe2_hbm, out_hbm, xe_v, xo_v, idx_v, *rest):
    rows = rest[:NBUF]
    gsem = rest[NBUF:2 * NBUF]
    ssem = rest[2 * NBUF:3 * NBUF]
    wid = lax.axis_index("s") * NC + lax.axis_index("c")
    base = wid * PPW
    r0 = wid * RPW

    pltpu.sync_copy(xe_hbm.at[pl.ds(base, PPW)], xe_v)
    pltpu.sync_copy(xo_hbm.at[pl.ds(base, PPW)], xo_v)

    iota10 = lax.iota(jnp.int32, 16) * np.int32(NJP)

    def cidx(s, carry):
        for u in range(UNROLL):
            t = s * UNROLL + u
            jp = t // (RPW // 16)
            tt = t % (RPW // 16)
            offs = iota10 + (tt * (16 * NJP) + jp)
            ge = plsc.load_gather(xe_v, [offs])
            go = plsc.load_gather(xo_v, [offs])
            pid = _round_clip(ge) * _RESF + _round_clip(go) - _P101
            idx_v[pl.ds(jp * RPW + tt * 16, 16)] = pid.astype(jnp.int32)
        return carry

    lax.fori_loop(0, PPW // (16 * UNROLL), cidx, 0)

    def fire_gather(g, h, b):
        pltpu.async_copy(
            pe2_hbm.at[idx_v.at[pl.ds(g * RPW + h * C, C)]], rows[b], gsem[b])

    def wait_gather(b):
        pltpu.make_async_copy(
            pe2_hbm.at[idx_v.at[pl.ds(0, C)]], rows[b], gsem[b]).wait()

    def fire_scatter(g, h, b):
        pltpu.async_copy(
            rows[b], out_hbm.at[g, pl.ds(r0 + h * C, C)], ssem[b])

    def wait_scatter(b):
        pltpu.make_async_copy(
            rows[b], out_hbm.at[0, pl.ds(0, C)], ssem[b]).wait()

    fire_gather(0, 0, 0)
    fire_gather(0, 1, 1)

    def group(g, carry):
        for b in range(NBUF):
            wait_gather(b)
            fire_scatter(g, b, b)
        for b in range(NBUF):
            wait_scatter(b)
            fire_gather(g + 1, b, b)
        return carry

    lax.fori_loop(0, NJP - 1, group, 0)

    for b in range(NBUF):
        wait_gather(b)
        fire_scatter(NJP - 1, b, b)
    for b in range(NBUF):
        wait_scatter(b)


@jax.jit
def _emb(xe, xo, pe2):
    mesh = plsc.VectorSubcoreMesh(core_axis_name="c", subcore_axis_name="s")
    k = pl.kernel(
        _body,
        out_type=jax.ShapeDtypeStruct((NJP, N_ROWS, 2 * D), jnp.float32),
        mesh=mesh,
        scratch_types=(
            [
                pltpu.VMEM((PPW,), jnp.float32),
                pltpu.VMEM((PPW,), jnp.float32),
                pltpu.VMEM((PPW,), jnp.int32),
            ]
            + [pltpu.VMEM((C, 2 * D), jnp.float32) for _ in range(NBUF)]
            + [pltpu.SemaphoreType.DMA for _ in range(2 * NBUF)]
        ),
        compiler_params=pltpu.CompilerParams(
            use_tc_tiling_on_sc=True, needs_layout_passes=False),
    )
    return k(xe, xo, pe2)


XR = 8192


def _xpose_body(in_ref, out_ref):
    a = in_ref[0]
    for sub in range(XR // 128):
        blk = a[sub * 128:(sub + 1) * 128, :]
        out_ref[:, :, sub * 128:(sub + 1) * 128] = blk.T.reshape(2, D, 128)


@jax.jit
def _xpose(mid):
    return pl.pallas_call(
        _xpose_body,
        grid=(NJP, N_ROWS // XR),
        in_specs=[pl.BlockSpec((1, XR, 128), lambda jp, rb: (jp, rb, 0))],
        out_specs=pl.BlockSpec((2, D, XR), lambda jp, rb: (jp, 0, rb)),
        out_shape=jax.ShapeDtypeStruct((N_COLS, D, N_ROWS), jnp.float32),
    )(mid)


def kernel(x, pe):
    xf = x.reshape(B)
    pe2 = jnp.concatenate(
        [
            jnp.repeat(pe, RES, axis=0),
            jnp.tile(pe, (RES, 1)),
        ],
        axis=1,
    )
    mid = _emb(xf[0::2], xf[1::2], pe2)
    out = _xpose(mid)
    return jnp.transpose(out, (2, 0, 1))


# speedup vs baseline: 3.7399x; 1.0046x over previous
import functools

import numpy as np

import jax
import jax.numpy as jnp
from jax import lax
from jax.experimental import pallas as pl
from jax.experimental.pallas import tpu as pltpu
from jax.experimental.pallas import tpu_sc as plsc

RES = 100
D = 64
N_ROWS = 16384
N_COLS = 20
NJP = N_COLS // 2
B = N_ROWS * N_COLS
P = B // 2
NC = 2
NS = 16
NW = NC * NS
RPW = N_ROWS // NW
PPW = P // NW
C = 256
NBUF = 2
UNROLL = 4

_MAGIC = np.float32(2.0 ** 23)
_LO = np.float32(1.0 / RES)
_ONE = np.float32(1.0)
_RESF = np.float32(RES)
_P101 = np.float32(101.0)


def _round_clip(v):
    xc = jnp.minimum(jnp.maximum(v, _LO), _ONE)
    return (xc * _RESF + _MAGIC) - _MAGIC


def _body(xe_hbm, xo_hbm, pe2_hbm, out_hbm, xe_v, xo_v, idx_v, *rest):
    rows = rest[:NBUF]
    gsem = rest[NBUF:2 * NBUF]
    ssem = rest[2 * NBUF:3 * NBUF]
    wid = lax.axis_index("s") * NC + lax.axis_index("c")
    base = wid * PPW
    r0 = wid * RPW

    pltpu.sync_copy(xe_hbm.at[pl.ds(base, PPW)], xe_v)
    pltpu.sync_copy(xo_hbm.at[pl.ds(base, PPW)], xo_v)

    iota10 = lax.iota(jnp.int32, 16) * np.int32(NJP)

    def cidx(s, carry):
        for u in range(UNROLL):
            t = s * UNROLL + u
            jp = t // (RPW // 16)
            tt = t % (RPW // 16)
            offs = iota10 + (tt * (16 * NJP) + jp)
            ge = plsc.load_gather(xe_v, [offs])
            go = plsc.load_gather(xo_v, [offs])
            pid = _round_clip(ge) * _RESF + _round_clip(go) - _P101
            idx_v[pl.ds(jp * RPW + tt * 16, 16)] = pid.astype(jnp.int32)
        return carry

    lax.fori_loop(0, PPW // (16 * UNROLL), cidx, 0)

    def fire_gather(g, h, b):
        pltpu.async_copy(
            pe2_hbm.at[idx_v.at[pl.ds(g * RPW + h * C, C)]], rows[b], gsem[b])

    def wait_gather(b):
        pltpu.make_async_copy(
            pe2_hbm.at[idx_v.at[pl.ds(0, C)]], rows[b], gsem[b]).wait()

    def fire_scatter(g, h, b):
        pltpu.async_copy(
            rows[b], out_hbm.at[g, pl.ds(r0 + h * C, C)], ssem[b])

    def wait_scatter(b):
        pltpu.make_async_copy(
            rows[b], out_hbm.at[0, pl.ds(0, C)], ssem[b]).wait()

    fire_gather(0, 0, 0)
    fire_gather(0, 1, 1)

    def group(g, carry):
        for b in range(NBUF):
            wait_gather(b)
            fire_scatter(g, b, b)
        for b in range(NBUF):
            wait_scatter(b)
            fire_gather(g + 1, b, b)
        return carry

    lax.fori_loop(0, NJP - 1, group, 0)

    for b in range(NBUF):
        wait_gather(b)
        fire_scatter(NJP - 1, b, b)
    for b in range(NBUF):
        wait_scatter(b)


@jax.jit
def _emb(xe, xo, pe2):
    mesh = plsc.VectorSubcoreMesh(core_axis_name="c", subcore_axis_name="s")
    k = pl.kernel(
        _body,
        out_type=jax.ShapeDtypeStruct((NJP, N_ROWS, 2 * D), jnp.float32),
        mesh=mesh,
        scratch_types=(
            [
                pltpu.VMEM((PPW,), jnp.float32),
                pltpu.VMEM((PPW,), jnp.float32),
                pltpu.VMEM((PPW,), jnp.int32),
            ]
            + [pltpu.VMEM((C, 2 * D), jnp.float32) for _ in range(NBUF)]
            + [pltpu.SemaphoreType.DMA for _ in range(2 * NBUF)]
        ),
        compiler_params=pltpu.CompilerParams(
            use_tc_tiling_on_sc=True, needs_layout_passes=False),
    )
    return k(xe, xo, pe2)


XR = 16384


def _xpose_body(in_ref, out_ref):
    a = in_ref[0]
    for sub in range(XR // 128):
        blk = a[sub * 128:(sub + 1) * 128, :]
        out_ref[:, :, sub * 128:(sub + 1) * 128] = blk.T.reshape(2, D, 128)


@jax.jit
def _xpose(mid):
    return pl.pallas_call(
        _xpose_body,
        grid=(NJP, N_ROWS // XR),
        in_specs=[pl.BlockSpec((1, XR, 128), lambda jp, rb: (jp, rb, 0))],
        out_specs=pl.BlockSpec((2, D, XR), lambda jp, rb: (jp, 0, rb)),
        out_shape=jax.ShapeDtypeStruct((N_COLS, D, N_ROWS), jnp.float32),
    )(mid)


def kernel(x, pe):
    xf = x.reshape(B)
    pe2 = jnp.concatenate(
        [
            jnp.repeat(pe, RES, axis=0),
            jnp.tile(pe, (RES, 1)),
        ],
        axis=1,
    )
    mid = _emb(xf[0::2], xf[1::2], pe2)
    out = _xpose(mid)
    return jnp.transpose(out, (2, 0, 1))
